# Initial kernel scaffold; baseline (speedup 1.0000x reference)
import jax
import jax.numpy as jnp
from jax.experimental import pallas as pl


def kernel(node_feature, hop_feature, nn_idx_f2v, nn_idx_v2f, efeature_f2v, efeature_v2f, params):
    raise NotImplementedError("write your pallas kernel here")


# speedup vs baseline: 22.5654x; 22.5654x over previous
import jax
import jax.numpy as jnp
from jax.experimental import pallas as pl

_B, _NV, _NF, _NFEAT, _HOP, _NETYPE = 512, 96, 48, 8, 6, 6
_DIMS = [64, 64, 64, 128, 128, 64, 64]
_SKIP = {3: 2, 4: 1, 5: 0}
_TILE = 16


def _vin_parts():
    parts = []
    for l in range(7):
        cur = _NFEAT if l == 0 else _DIMS[l - 1]
        ps = [cur]
        if l in _SKIP:
            ps.append(_DIMS[_SKIP[l]])
        parts.append(ps)
    return parts


_VPS = _vin_parts()


def _build_wnames():
    names = []
    for nm in ['v2f', 'f2v']:
        names += ['em_%s_W1T' % nm, 'em_%s_b1' % nm, 'em_%s_W2T' % nm, 'em_%s_b2' % nm]
    for l in range(7):
        for g in range(2):
            for j in range(len(_VPS[l])):
                names.append('mpv2f_W_%d_%d_p%d' % (l, g, j))
            names += ['mpv2f_b_%d_%d' % (l, g), 'mpv2f_S_%d_%d' % (l, g),
                      'fc_Wmv_%d_%d' % (l, g), 'fc_Wf_%d_%d' % (l, g), 'fc_b_%d_%d' % (l, g),
                      'mpf2v_W_%d_%d' % (l, g), 'mpf2v_b_%d_%d' % (l, g), 'mpf2v_S_%d_%d' % (l, g)]
        names += ['vc_Wm0_%d' % l, 'vc_Wm1_%d' % l]
        for j in range(len(_VPS[l])):
            names.append('vc_Wv_%d_p%d' % (l, j))
        names.append('vc_b_%d' % l)
    names += ['out_WT', 'out_b']
    return names


_WNAMES = _build_wnames()
_RGNAMES = ['rg_W1T', 'rg_b1', 'rg_gamma', 'rg_beta', 'rg_W2T', 'rg_b2', 'rg_W3T', 'rg_b3']


def _prep(p):
    w = {}
    for nm in ['f2v', 'v2f']:
        w['em_%s_W1T' % nm] = p['em_' + nm + '_W1'].T
        w['em_%s_b1' % nm] = p['em_' + nm + '_b1'][None, :]
        w['em_%s_W2T' % nm] = p['em_' + nm + '_W2'].T
        w['em_%s_b2' % nm] = p['em_' + nm + '_b2'][None, :]
    for l in range(7):
        d = _DIMS[l]
        ps = _VPS[l]
        for g in range(2):
            W = p['mpv2f_W_%d_%d' % (l, g)]
            o = 0
            for j, c in enumerate(ps):
                w['mpv2f_W_%d_%d_p%d' % (l, g, j)] = W[:, o:o + c].T
                o += c
            w['mpv2f_b_%d_%d' % (l, g)] = p['mpv2f_b_%d_%d' % (l, g)][None, :]
            w['mpv2f_S_%d_%d' % (l, g)] = p['mpv2f_S_%d_%d' % (l, g)]
            fi = ([_HOP, _NV][g]) if l == 0 else _DIMS[l - 1]
            Wfc = p['fc_W_%d_%d' % (l, g)]
            w['fc_Wmv_%d_%d' % (l, g)] = Wfc[:, :d].T
            w['fc_Wf_%d_%d' % (l, g)] = Wfc[:, d:d + fi].T
            w['fc_b_%d_%d' % (l, g)] = p['fc_b_%d_%d' % (l, g)][None, :]
            w['mpf2v_W_%d_%d' % (l, g)] = p['mpf2v_W_%d_%d' % (l, g)].T
            w['mpf2v_b_%d_%d' % (l, g)] = p['mpf2v_b_%d_%d' % (l, g)][None, :]
            w['mpf2v_S_%d_%d' % (l, g)] = p['mpf2v_S_%d_%d' % (l, g)]
        Wvc = p['vc_W_%d' % l]
        w['vc_Wm0_%d' % l] = Wvc[:, :d].T
        w['vc_Wm1_%d' % l] = Wvc[:, d:2 * d].T
        o = 2 * d
        for j, c in enumerate(ps):
            w['vc_Wv_%d_p%d' % (l, j)] = Wvc[:, o:o + c].T
            o += c
        w['vc_b_%d' % l] = p['vc_b_%d' % l][None, :]
    w['out_WT'] = p['out_W'].T
    w['out_b'] = p['out_b'][None, :]
    w['rg_W1T'] = p['rg_W1'].T
    w['rg_W2T'] = p['rg_W2'].T
    w['rg_W3T'] = p['rg_W3'].T
    for nm in ['rg_b1', 'rg_b2', 'rg_b3', 'rg_gamma', 'rg_beta']:
        w[nm] = p[nm][None, :]
    return [w[n] for n in _WNAMES], [w[n] for n in _RGNAMES]


def _mm(a, b):
    return jnp.dot(a, b, preferred_element_type=jnp.float32)


def _bmm(oh, x):
    return jax.lax.dot_general(oh, x, (((2,), (1,)), ((0,), (0,))),
                               preferred_element_type=jnp.float32,
                               precision=jax.lax.Precision.HIGHEST)


def _fwd_body(*refs):
    T = _TILE
    nf_ref, hop_ref, nch0f_ref, iv_ref, if_ref, efv_ref, eff_ref = refs[:7]
    wrefs = refs[7:7 + len(_WNAMES)]
    res_ref, hhop_ref = refs[7 + len(_WNAMES):]
    w = {nm: r[...] for nm, r in zip(_WNAMES, wrefs)}

    nf = nf_ref[...]
    nch0f = nch0f_ref[...]

    efv_f = efv_ref[...].reshape(6 * T * _NF, 7)
    eff_f = eff_ref[...].reshape(3 * T * _NV, 7)
    ETv = _mm(jax.nn.relu(_mm(efv_f, w['em_v2f_W1T']) + w['em_v2f_b1']), w['em_v2f_W2T']) + w['em_v2f_b2']
    ETf = _mm(jax.nn.relu(_mm(eff_f, w['em_f2v_W1T']) + w['em_f2v_b1']), w['em_f2v_W2T']) + w['em_f2v_b2']

    iv = iv_ref[...]
    if_ = if_ref[...]
    iota_v = jax.lax.broadcasted_iota(jnp.int32, (T, _NF, _NV), 2)
    iota_f = jax.lax.broadcasted_iota(jnp.int32, (T, _NV, _NF), 2)
    oh_v2f = [(iv[:, :, k][:, :, None] == iota_v).astype(jnp.float32) for k in range(6)]
    oh_f2v = [(if_[:, :, k][:, :, None] == iota_f).astype(jnp.float32) for k in range(3)]

    def mm_parts(parts, pref):
        acc = None
        for j, x in enumerate(parts):
            t = _mm(x, w['%s_p%d' % (pref, j)])
            acc = t if acc is None else acc + t
        return acc

    var_parts = [nf.reshape(T * _NV, _NFEAT)]
    fac0 = hop_ref[...].reshape(T * _NF, _HOP)
    fac1 = nf[:, :, 0]
    outs = []
    for l in range(7):
        d = _DIMS[l]
        if l in _SKIP:
            var_parts = [var_parts[0], outs[_SKIP[l]]]
        Wv0 = (mm_parts(var_parts, 'mpv2f_W_%d_0' % l) + w['mpv2f_b_%d_0' % l]).reshape(T, _NV, d)
        gate_v = jax.nn.sigmoid(_mm(ETv, w['mpv2f_S_%d_0' % l]))
        mv0 = None
        for k in range(6):
            hk = _bmm(oh_v2f[k], Wv0).reshape(T * _NF, d) * gate_v[k * T * _NF:(k + 1) * T * _NF]
            mv0 = hk if mv0 is None else jnp.maximum(mv0, hk)
        Wv1 = (mm_parts(var_parts, 'mpv2f_W_%d_1' % l) + w['mpv2f_b_%d_1' % l]).reshape(T, _NV, d)
        mv1 = jnp.max(Wv1, axis=1) * jax.nn.sigmoid(w['mpv2f_S_%d_1' % l])
        fg0 = jax.nn.relu(_mm(mv0, w['fc_Wmv_%d_0' % l]) + _mm(fac0, w['fc_Wf_%d_0' % l]) + w['fc_b_%d_0' % l])
        fg1 = jax.nn.relu(_mm(mv1, w['fc_Wmv_%d_1' % l]) + _mm(fac1, w['fc_Wf_%d_1' % l]) + w['fc_b_%d_1' % l])
        Wf0 = (_mm(fg0, w['mpf2v_W_%d_0' % l]) + w['mpf2v_b_%d_0' % l]).reshape(T, _NF, d)
        gate_f = jax.nn.sigmoid(_mm(ETf, w['mpf2v_S_%d_0' % l]))
        msg0 = None
        for k in range(3):
            hk = _bmm(oh_f2v[k], Wf0).reshape(T * _NV, d) * gate_f[k * T * _NV:(k + 1) * T * _NV]
            msg0 = hk if msg0 is None else jnp.maximum(msg0, hk)
        m1 = (_mm(fg1, w['mpf2v_W_%d_1' % l]) + w['mpf2v_b_%d_1' % l]) * jax.nn.sigmoid(w['mpf2v_S_%d_1' % l])
        acc = _mm(msg0, w['vc_Wm0_%d' % l])
        for j, x in enumerate(var_parts):
            acc = acc + _mm(x, w['vc_Wv_%d_p%d' % (l, j)])
        m1w = _mm(m1, w['vc_Wm1_%d' % l])
        var_new = jax.nn.relu(acc.reshape(T, _NV, d) + m1w[:, None, :] + w['vc_b_%d' % l])
        var_new = var_new.reshape(T * _NV, d)
        fac0, fac1 = fg0, fg1
        outs.append(var_new)
        var_parts = [var_new]
    res_ref[...] = _mm(outs[-1], w['out_WT']) + w['out_b'] + nch0f
    hhop_ref[...] = fac1


def _rg_body(hhop_ref, w1t, b1, gamma, beta, w2t, b2, w3t, b3, out_ref):
    h = _mm(hhop_ref[...], w1t[...]) + b1[...]
    mu = jnp.mean(h, axis=0, keepdims=True)
    va = jnp.mean((h - mu) ** 2, axis=0, keepdims=True)
    h = gamma[...] * (h - mu) / jnp.sqrt(va + 1e-5) + beta[...]
    h = jax.nn.relu(h)
    h = jax.nn.relu(_mm(h, w2t[...]) + b2[...])
    out_ref[...] = jax.nn.relu(_mm(h, w3t[...]) + b3[...])


def kernel(node_feature, hop_feature, nn_idx_f2v, nn_idx_v2f, efeature_f2v, efeature_v2f, params):
    T = _TILE
    f32 = jnp.float32
    nf = jnp.squeeze(node_feature, -1).transpose(0, 2, 1)
    hop = jnp.squeeze(hop_feature, -1).transpose(0, 2, 1)
    nch0f = node_feature[:, 0, :, :].reshape(_B * _NV, 1)
    iv = nn_idx_v2f.astype(jnp.int32)
    if_ = nn_idx_f2v.astype(jnp.int32)
    efv = efeature_v2f.transpose(3, 0, 2, 1)
    eff = efeature_f2v.transpose(3, 0, 2, 1)
    wlist, rglist = _prep(params)

    const = lambda a: pl.BlockSpec(a.shape, lambda i: (0,) * a.ndim)
    in_specs = [
        pl.BlockSpec((T, _NV, _NFEAT), lambda i: (i, 0, 0)),
        pl.BlockSpec((T, _NF, _HOP), lambda i: (i, 0, 0)),
        pl.BlockSpec((T * _NV, 1), lambda i: (i, 0)),
        pl.BlockSpec((T, _NF, 6), lambda i: (i, 0, 0)),
        pl.BlockSpec((T, _NV, 3), lambda i: (i, 0, 0)),
        pl.BlockSpec((6, T, _NF, 7), lambda i: (0, i, 0, 0)),
        pl.BlockSpec((3, T, _NV, 7), lambda i: (0, i, 0, 0)),
    ] + [const(a) for a in wlist]
    out_specs = [
        pl.BlockSpec((T * _NV, 1), lambda i: (i, 0)),
        pl.BlockSpec((T, 64), lambda i: (i, 0)),
    ]
    res_f, hhop = pl.pallas_call(
        _fwd_body,
        grid=(_B // T,),
        in_specs=in_specs,
        out_specs=out_specs,
        out_shape=[jax.ShapeDtypeStruct((_B * _NV, 1), f32),
                   jax.ShapeDtypeStruct((_B, 64), f32)],
    )(nf, hop, nch0f, iv, if_, efv, eff, *wlist)

    snr = pl.pallas_call(
        _rg_body,
        in_specs=[pl.BlockSpec(a.shape, lambda: (0,) * a.ndim) for a in [hhop] + rglist],
        out_specs=pl.BlockSpec((_B, 1), lambda: (0, 0)),
        out_shape=jax.ShapeDtypeStruct((_B, 1), f32),
    )(hhop, *rglist)

    res = res_f.reshape(_B, _NV)[:, :_NF]
    return res, snr


# speedup vs baseline: 46.0448x; 2.0405x over previous
import jax
import jax.numpy as jnp
from jax.experimental import pallas as pl

_B, _NV, _NF, _NFEAT, _HOP, _NETYPE = 512, 96, 48, 8, 6, 6
_DIMS = [64, 64, 64, 128, 128, 64, 64]
_SKIP = {3: 2, 4: 1, 5: 0}
_TILE = 16


def _vin_parts():
    parts = []
    for l in range(7):
        cur = _NFEAT if l == 0 else _DIMS[l - 1]
        ps = [cur]
        if l in _SKIP:
            ps.append(_DIMS[_SKIP[l]])
        parts.append(ps)
    return parts


_VPS = _vin_parts()


def _build_wnames():
    names = []
    for nm in ['v2f', 'f2v']:
        names += ['em_%s_W1T' % nm, 'em_%s_b1' % nm, 'em_%s_W2T' % nm, 'em_%s_b2' % nm]
    for l in range(7):
        for j in range(len(_VPS[l])):
            names.append('mpv2f_W_%d_01_p%d' % (l, j))
        names.append('mpv2f_b_%d_01' % l)
        for g in range(2):
            names += ['mpv2f_S_%d_%d' % (l, g),
                      'fc_Wmv_%d_%d' % (l, g), 'fc_Wf_%d_%d' % (l, g), 'fc_b_%d_%d' % (l, g),
                      'mpf2v_W_%d_%d' % (l, g), 'mpf2v_b_%d_%d' % (l, g), 'mpf2v_S_%d_%d' % (l, g)]
        names += ['vc_Wm0_%d' % l, 'vc_Wm1_%d' % l]
        for j in range(len(_VPS[l])):
            names.append('vc_Wv_%d_p%d' % (l, j))
        names.append('vc_b_%d' % l)
    names += ['out_WT', 'out_b']
    return names


_WNAMES = _build_wnames()
_RGNAMES = ['rg_W1T', 'rg_b1', 'rg_gamma', 'rg_beta', 'rg_W2T', 'rg_b2', 'rg_W3T', 'rg_b3']


def _prep(p):
    w = {}
    for nm in ['f2v', 'v2f']:
        w['em_%s_W1T' % nm] = p['em_' + nm + '_W1'].T
        w['em_%s_b1' % nm] = p['em_' + nm + '_b1'][None, :]
        w['em_%s_W2T' % nm] = p['em_' + nm + '_W2'].T
        w['em_%s_b2' % nm] = p['em_' + nm + '_b2'][None, :]
    for l in range(7):
        d = _DIMS[l]
        ps = _VPS[l]
        W0, W1 = p['mpv2f_W_%d_0' % l], p['mpv2f_W_%d_1' % l]
        o = 0
        for j, c in enumerate(ps):
            w['mpv2f_W_%d_01_p%d' % (l, j)] = jnp.concatenate(
                [W0[:, o:o + c].T, W1[:, o:o + c].T], axis=1)
            o += c
        w['mpv2f_b_%d_01' % l] = jnp.concatenate(
            [p['mpv2f_b_%d_0' % l], p['mpv2f_b_%d_1' % l]])[None, :]
        for g in range(2):
            w['mpv2f_S_%d_%d' % (l, g)] = p['mpv2f_S_%d_%d' % (l, g)]
            fi = ([_HOP, _NV][g]) if l == 0 else _DIMS[l - 1]
            Wfc = p['fc_W_%d_%d' % (l, g)]
            w['fc_Wmv_%d_%d' % (l, g)] = Wfc[:, :d].T
            w['fc_Wf_%d_%d' % (l, g)] = Wfc[:, d:d + fi].T
            w['fc_b_%d_%d' % (l, g)] = p['fc_b_%d_%d' % (l, g)][None, :]
            w['mpf2v_W_%d_%d' % (l, g)] = p['mpf2v_W_%d_%d' % (l, g)].T
            w['mpf2v_b_%d_%d' % (l, g)] = p['mpf2v_b_%d_%d' % (l, g)][None, :]
            w['mpf2v_S_%d_%d' % (l, g)] = p['mpf2v_S_%d_%d' % (l, g)]
        Wvc = p['vc_W_%d' % l]
        w['vc_Wm0_%d' % l] = Wvc[:, :d].T
        w['vc_Wm1_%d' % l] = Wvc[:, d:2 * d].T
        o = 2 * d
        for j, c in enumerate(ps):
            w['vc_Wv_%d_p%d' % (l, j)] = Wvc[:, o:o + c].T
            o += c
        w['vc_b_%d' % l] = p['vc_b_%d' % l][None, :]
    w['out_WT'] = p['out_W'].T
    w['out_b'] = p['out_b'][None, :]
    w['rg_W1T'] = p['rg_W1'].T
    w['rg_W2T'] = p['rg_W2'].T
    w['rg_W3T'] = p['rg_W3'].T
    for nm in ['rg_b1', 'rg_b2', 'rg_b3', 'rg_gamma', 'rg_beta']:
        w[nm] = p[nm][None, :]
    return [w[n] for n in _WNAMES], [w[n] for n in _RGNAMES]


def _mm(a, b):
    return jnp.dot(a, b, preferred_element_type=jnp.float32)


def _bmm_raw(oh, x):
    return jax.lax.dot_general(oh, x, (((2,), (1,)), ((0,), (0,))),
                               preferred_element_type=jnp.float32)


def _bmm(oh, x):
    hi = x.astype(jnp.bfloat16).astype(jnp.float32)
    return _bmm_raw(oh, hi) + _bmm_raw(oh, x - hi)


def _fwd_body(*refs):
    T = _TILE
    nf_ref, hop_ref, nch0f_ref, iv_ref, if_ref, efv_ref, eff_ref = refs[:7]
    wrefs = refs[7:7 + len(_WNAMES)]
    res_ref, hhop_ref = refs[7 + len(_WNAMES):]
    w = {nm: r[...] for nm, r in zip(_WNAMES, wrefs)}

    nf = nf_ref[...]
    nch0f = nch0f_ref[...]

    efv_f = efv_ref[...].reshape(6 * T * _NF, 7)
    eff_f = eff_ref[...].reshape(3 * T * _NV, 7)
    ETv = _mm(jax.nn.relu(_mm(efv_f, w['em_v2f_W1T']) + w['em_v2f_b1']), w['em_v2f_W2T']) + w['em_v2f_b2']
    ETf = _mm(jax.nn.relu(_mm(eff_f, w['em_f2v_W1T']) + w['em_f2v_b1']), w['em_f2v_W2T']) + w['em_f2v_b2']

    iv = iv_ref[...]
    if_ = if_ref[...]
    iota_v = jax.lax.broadcasted_iota(jnp.int32, (T, _NF, _NV), 2)
    iota_f = jax.lax.broadcasted_iota(jnp.int32, (T, _NV, _NF), 2)
    oh_v2f = [(iv[:, :, k][:, :, None] == iota_v).astype(jnp.float32) for k in range(6)]
    oh_f2v = [(if_[:, :, k][:, :, None] == iota_f).astype(jnp.float32) for k in range(3)]

    def mm_parts(parts, pref):
        acc = None
        for j, x in enumerate(parts):
            t = _mm(x, w['%s_p%d' % (pref, j)])
            acc = t if acc is None else acc + t
        return acc

    var_parts = [nf.reshape(T * _NV, _NFEAT)]
    fac0 = hop_ref[...].reshape(T * _NF, _HOP)
    fac1 = nf[:, :, 0]
    outs = []
    for l in range(7):
        d = _DIMS[l]
        if l in _SKIP:
            var_parts = [var_parts[0], outs[_SKIP[l]]]
        Wv01 = (mm_parts(var_parts, 'mpv2f_W_%d_01' % l) + w['mpv2f_b_%d_01' % l]).reshape(T, _NV, 2 * d)
        Wv0 = Wv01[:, :, :d]
        gate_v = jax.nn.sigmoid(_mm(ETv, w['mpv2f_S_%d_0' % l]))
        mv0 = None
        for k in range(6):
            hk = _bmm(oh_v2f[k], Wv0).reshape(T * _NF, d) * gate_v[k * T * _NF:(k + 1) * T * _NF]
            mv0 = hk if mv0 is None else jnp.maximum(mv0, hk)
        mv1 = jnp.max(Wv01[:, :, d:], axis=1) * jax.nn.sigmoid(w['mpv2f_S_%d_1' % l])
        fg0 = jax.nn.relu(_mm(mv0, w['fc_Wmv_%d_0' % l]) + _mm(fac0, w['fc_Wf_%d_0' % l]) + w['fc_b_%d_0' % l])
        fg1 = jax.nn.relu(_mm(mv1, w['fc_Wmv_%d_1' % l]) + _mm(fac1, w['fc_Wf_%d_1' % l]) + w['fc_b_%d_1' % l])
        Wf0 = (_mm(fg0, w['mpf2v_W_%d_0' % l]) + w['mpf2v_b_%d_0' % l]).reshape(T, _NF, d)
        gate_f = jax.nn.sigmoid(_mm(ETf, w['mpf2v_S_%d_0' % l]))
        msg0 = None
        for k in range(3):
            hk = _bmm(oh_f2v[k], Wf0).reshape(T * _NV, d) * gate_f[k * T * _NV:(k + 1) * T * _NV]
            msg0 = hk if msg0 is None else jnp.maximum(msg0, hk)
        m1 = (_mm(fg1, w['mpf2v_W_%d_1' % l]) + w['mpf2v_b_%d_1' % l]) * jax.nn.sigmoid(w['mpf2v_S_%d_1' % l])
        acc = _mm(msg0, w['vc_Wm0_%d' % l])
        for j, x in enumerate(var_parts):
            acc = acc + _mm(x, w['vc_Wv_%d_p%d' % (l, j)])
        m1w = _mm(m1, w['vc_Wm1_%d' % l])
        var_new = jax.nn.relu(acc.reshape(T, _NV, d) + m1w[:, None, :] + w['vc_b_%d' % l])
        var_new = var_new.reshape(T * _NV, d)
        fac0, fac1 = fg0, fg1
        outs.append(var_new)
        var_parts = [var_new]
    res_ref[...] = _mm(outs[-1], w['out_WT']) + w['out_b'] + nch0f
    hhop_ref[...] = fac1


def _rg_body(hhop_ref, w1t, b1, gamma, beta, w2t, b2, w3t, b3, out_ref):
    h = _mm(hhop_ref[...], w1t[...]) + b1[...]
    mu = jnp.mean(h, axis=0, keepdims=True)
    va = jnp.mean((h - mu) ** 2, axis=0, keepdims=True)
    h = gamma[...] * (h - mu) / jnp.sqrt(va + 1e-5) + beta[...]
    h = jax.nn.relu(h)
    h = jax.nn.relu(_mm(h, w2t[...]) + b2[...])
    out_ref[...] = jax.nn.relu(_mm(h, w3t[...]) + b3[...])


def kernel(node_feature, hop_feature, nn_idx_f2v, nn_idx_v2f, efeature_f2v, efeature_v2f, params):
    T = _TILE
    f32 = jnp.float32
    nf = jnp.squeeze(node_feature, -1).transpose(0, 2, 1)
    hop = jnp.squeeze(hop_feature, -1).transpose(0, 2, 1)
    nch0f = node_feature[:, 0, :, :].reshape(_B * _NV, 1)
    iv = nn_idx_v2f.astype(jnp.int32)
    if_ = nn_idx_f2v.astype(jnp.int32)
    efv = efeature_v2f.transpose(3, 0, 2, 1)
    eff = efeature_f2v.transpose(3, 0, 2, 1)
    wlist, rglist = _prep(params)

    const = lambda a: pl.BlockSpec(a.shape, lambda i: (0,) * a.ndim)
    in_specs = [
        pl.BlockSpec((T, _NV, _NFEAT), lambda i: (i, 0, 0)),
        pl.BlockSpec((T, _NF, _HOP), lambda i: (i, 0, 0)),
        pl.BlockSpec((T * _NV, 1), lambda i: (i, 0)),
        pl.BlockSpec((T, _NF, 6), lambda i: (i, 0, 0)),
        pl.BlockSpec((T, _NV, 3), lambda i: (i, 0, 0)),
        pl.BlockSpec((6, T, _NF, 7), lambda i: (0, i, 0, 0)),
        pl.BlockSpec((3, T, _NV, 7), lambda i: (0, i, 0, 0)),
    ] + [const(a) for a in wlist]
    out_specs = [
        pl.BlockSpec((T * _NV, 1), lambda i: (i, 0)),
        pl.BlockSpec((T, 64), lambda i: (i, 0)),
    ]
    res_f, hhop = pl.pallas_call(
        _fwd_body,
        grid=(_B // T,),
        in_specs=in_specs,
        out_specs=out_specs,
        out_shape=[jax.ShapeDtypeStruct((_B * _NV, 1), f32),
                   jax.ShapeDtypeStruct((_B, 64), f32)],
    )(nf, hop, nch0f, iv, if_, efv, eff, *wlist)

    snr = pl.pallas_call(
        _rg_body,
        in_specs=[pl.BlockSpec(a.shape, lambda: (0,) * a.ndim) for a in [hhop] + rglist],
        out_specs=pl.BlockSpec((_B, 1), lambda: (0, 0)),
        out_shape=jax.ShapeDtypeStruct((_B, 1), f32),
    )(hhop, *rglist)

    res = res_f.reshape(_B, _NV)[:, :_NF]
    return res, snr
